# Initial kernel scaffold; baseline (speedup 1.0000x reference)
import jax
import jax.numpy as jnp
from jax.experimental import pallas as pl


def kernel(x, Wl, Wr, att, bias):
    raise NotImplementedError("write your pallas kernel here")


# speedup vs baseline: 6.2164x; 6.2164x over previous
import functools

import jax
import jax.numpy as jnp
from jax import lax
from jax.experimental import pallas as pl
from jax.experimental.pallas import tpu as pltpu
from jax.experimental.pallas import tpu_sc as plsc

_N = 10000
_D = 256
_C = 256
_K = 16

_R1 = 200
_R3 = 200
_GBLK = 128


def _knn_proj_body(x_blk, xt_ref, wl_ref, wr_ref, hl_out, hr_out, idx_out):
    rows = x_blk[...]
    b = pl.program_id(0)
    hl_out[...] = jnp.dot(rows, wl_ref[...], preferred_element_type=jnp.float32)
    hr_out[...] = jnp.dot(rows, wr_ref[...], preferred_element_type=jnp.float32)

    xt = xt_ref[...]
    sq = jnp.sum(xt * xt, axis=0, keepdims=True)
    rsq = jnp.sum(rows * rows, axis=1, keepdims=True)
    prod = jnp.dot(rows, xt, preferred_element_type=jnp.float32)
    d2 = rsq - 2.0 * prod + sq

    cols = lax.broadcasted_iota(jnp.int32, (_R1, _N), 1)
    row_ids = b * _R1 + lax.broadcasted_iota(jnp.int32, (_R1, 1), 0)
    d2 = jnp.where(cols == row_ids, jnp.inf, d2)

    picks = []
    for _ in range(_K):
        m = jnp.min(d2, axis=1, keepdims=True)
        i = jnp.min(jnp.where(d2 == m, cols, _N), axis=1, keepdims=True)
        picks.append(i)
        d2 = jnp.where(cols == i, jnp.inf, d2)
    idx_out[...] = jnp.concatenate(picks, axis=1)


def _stage1(x, xt, wl, wr):
    return pl.pallas_call(
        _knn_proj_body,
        grid=(_N // _R1,),
        in_specs=[
            pl.BlockSpec((_R1, _D), lambda b: (b, 0)),
            pl.BlockSpec((_D, _N), lambda b: (0, 0)),
            pl.BlockSpec((_D, _C), lambda b: (0, 0)),
            pl.BlockSpec((_D, _C), lambda b: (0, 0)),
        ],
        out_specs=[
            pl.BlockSpec((_R1, _C), lambda b: (b, 0)),
            pl.BlockSpec((_R1, _C), lambda b: (b, 0)),
            pl.BlockSpec((_R1, _K), lambda b: (b, 0)),
        ],
        out_shape=[
            jax.ShapeDtypeStruct((_N, _C), jnp.float32),
            jax.ShapeDtypeStruct((_N, _C), jnp.float32),
            jax.ShapeDtypeStruct((_N, _K), jnp.int32),
        ],
    )(x, xt, wl, wr)


_NBLOCKS = (_N * _K) // _GBLK
_NW = 32


def _sc_gather(hl, idx_flat):
    mesh = plsc.VectorSubcoreMesh(core_axis_name="c", subcore_axis_name="s")

    @functools.partial(
        pl.kernel,
        mesh=mesh,
        out_type=jax.ShapeDtypeStruct((_N * _K, _C), jnp.float32),
        scratch_types=[
            pltpu.VMEM((_GBLK,), jnp.int32),
            pltpu.VMEM((_GBLK, _C), jnp.float32),
            pltpu.SemaphoreType.DMA,
        ],
    )
    def gather_kernel(hl_hbm, idx_hbm, out_hbm, idx_v, rows_v, sem):
        cid = lax.axis_index("c")
        sid = lax.axis_index("s")
        w = sid * 2 + cid
        nb_w = (_NBLOCKS - w + _NW - 1) // _NW

        def body(i, carry):
            blk = w + i * _NW
            pltpu.sync_copy(idx_hbm.at[pl.ds(blk * _GBLK, _GBLK)], idx_v)
            pltpu.async_copy(hl_hbm.at[idx_v], rows_v, sem).wait()
            pltpu.sync_copy(rows_v, out_hbm.at[pl.ds(blk * _GBLK, _GBLK)])
            return carry

        lax.fori_loop(0, nb_w, body, 0)

    return gather_kernel(hl, idx_flat)


def _attn_body(g_ref, hr_ref, att_ref, bias_ref, out_ref):
    g = g_ref[...]
    hr = hr_ref[...]
    a = att_ref[...]
    es = []
    for k in range(_K):
        gk = g[:, k * _C:(k + 1) * _C]
        z = gk + hr
        z = jnp.where(z > 0, z, 0.2 * z)
        es.append(jnp.sum(z * a, axis=1, keepdims=True))
    e = jnp.concatenate(es, axis=1)
    m = jnp.max(e, axis=1, keepdims=True)
    ez = jnp.exp(e - m)
    denom = jnp.sum(ez, axis=1, keepdims=True)
    alpha = ez / denom
    acc = alpha[:, 0:1] * g[:, 0:_C]
    for k in range(1, _K):
        acc = acc + alpha[:, k:k + 1] * g[:, k * _C:(k + 1) * _C]
    out_ref[...] = acc + bias_ref[...]


def _stage3(g2, hr, att2, bias2):
    return pl.pallas_call(
        _attn_body,
        grid=(_N // _R3,),
        in_specs=[
            pl.BlockSpec((_R3, _K * _C), lambda b: (b, 0)),
            pl.BlockSpec((_R3, _C), lambda b: (b, 0)),
            pl.BlockSpec((1, _C), lambda b: (0, 0)),
            pl.BlockSpec((1, _C), lambda b: (0, 0)),
        ],
        out_specs=pl.BlockSpec((_R3, _C), lambda b: (b, 0)),
        out_shape=jax.ShapeDtypeStruct((_N, _C), jnp.float32),
    )(g2, hr, att2, bias2)


def kernel(x, Wl, Wr, att, bias):
    xt = x.T
    hl, hr, idx = _stage1(x, xt, Wl, Wr)
    g = _sc_gather(hl, idx.reshape(-1))
    g2 = g.reshape(_N, _K * _C)
    out = _stage3(g2, hr, att.reshape(1, _C), bias.reshape(1, _C))
    return out


# speedup vs baseline: 6.6111x; 1.0635x over previous
import functools

import jax
import jax.numpy as jnp
from jax import lax
from jax.experimental import pallas as pl
from jax.experimental.pallas import tpu as pltpu
from jax.experimental.pallas import tpu_sc as plsc

_N = 10000
_D = 256
_C = 256
_K = 16

_R1 = 200
_R3 = 200
_GBLK = 128


def _knn_proj_body(x_blk, xt_ref, wl_ref, wr_ref, hl_out, hr_out, idx_out):
    rows = x_blk[...]
    b = pl.program_id(0)
    hl_out[...] = jnp.dot(rows, wl_ref[...], preferred_element_type=jnp.float32)
    hr_out[...] = jnp.dot(rows, wr_ref[...], preferred_element_type=jnp.float32)

    xt = xt_ref[...]
    sq = jnp.sum(xt * xt, axis=0, keepdims=True)
    prod = jnp.dot(rows, xt, preferred_element_type=jnp.float32)
    d2 = sq - 2.0 * prod

    cols = lax.broadcasted_iota(jnp.int32, (_R1, _N), 1)
    row_ids = b * _R1 + lax.broadcasted_iota(jnp.int32, (_R1, 1), 0)
    d2 = jnp.where(cols == row_ids, jnp.inf, d2)

    picks = []
    for _ in range(_K):
        i = jnp.argmin(d2, axis=1).astype(jnp.int32)[:, None]
        picks.append(i)
        d2 = jnp.where(cols == i, jnp.inf, d2)
    idx_out[...] = jnp.concatenate(picks, axis=1)


def _stage1(x, xt, wl, wr):
    return pl.pallas_call(
        _knn_proj_body,
        grid=(_N // _R1,),
        in_specs=[
            pl.BlockSpec((_R1, _D), lambda b: (b, 0)),
            pl.BlockSpec((_D, _N), lambda b: (0, 0)),
            pl.BlockSpec((_D, _C), lambda b: (0, 0)),
            pl.BlockSpec((_D, _C), lambda b: (0, 0)),
        ],
        out_specs=[
            pl.BlockSpec((_R1, _C), lambda b: (b, 0)),
            pl.BlockSpec((_R1, _C), lambda b: (b, 0)),
            pl.BlockSpec((_R1, _K), lambda b: (b, 0)),
        ],
        out_shape=[
            jax.ShapeDtypeStruct((_N, _C), jnp.float32),
            jax.ShapeDtypeStruct((_N, _C), jnp.float32),
            jax.ShapeDtypeStruct((_N, _K), jnp.int32),
        ],
    )(x, xt, wl, wr)


_NBLOCKS = (_N * _K) // _GBLK
_NW = 32


def _sc_gather(hl, idx_flat):
    mesh = plsc.VectorSubcoreMesh(core_axis_name="c", subcore_axis_name="s")

    @functools.partial(
        pl.kernel,
        mesh=mesh,
        out_type=jax.ShapeDtypeStruct((_N * _K, _C), jnp.float32),
        scratch_types=[
            pltpu.VMEM((_GBLK,), jnp.int32),
            pltpu.VMEM((_GBLK, _C), jnp.float32),
            pltpu.SemaphoreType.DMA,
        ],
    )
    def gather_kernel(hl_hbm, idx_hbm, out_hbm, idx_v, rows_v, sem):
        cid = lax.axis_index("c")
        sid = lax.axis_index("s")
        w = sid * 2 + cid
        nb_w = (_NBLOCKS - w + _NW - 1) // _NW

        def body(i, carry):
            blk = w + i * _NW
            pltpu.sync_copy(idx_hbm.at[pl.ds(blk * _GBLK, _GBLK)], idx_v)
            pltpu.async_copy(hl_hbm.at[idx_v], rows_v, sem).wait()
            pltpu.sync_copy(rows_v, out_hbm.at[pl.ds(blk * _GBLK, _GBLK)])
            return carry

        lax.fori_loop(0, nb_w, body, 0)

    return gather_kernel(hl, idx_flat)


def _attn_body(g_ref, hr_ref, att_ref, bias_ref, out_ref):
    g = g_ref[...]
    hr = hr_ref[...]
    a = att_ref[...]
    es = []
    for k in range(_K):
        gk = g[:, k * _C:(k + 1) * _C]
        z = gk + hr
        z = jnp.where(z > 0, z, 0.2 * z)
        es.append(jnp.sum(z * a, axis=1, keepdims=True))
    e = jnp.concatenate(es, axis=1)
    m = jnp.max(e, axis=1, keepdims=True)
    ez = jnp.exp(e - m)
    denom = jnp.sum(ez, axis=1, keepdims=True)
    alpha = ez / denom
    acc = alpha[:, 0:1] * g[:, 0:_C]
    for k in range(1, _K):
        acc = acc + alpha[:, k:k + 1] * g[:, k * _C:(k + 1) * _C]
    out_ref[...] = acc + bias_ref[...]


def _stage3(g2, hr, att2, bias2):
    return pl.pallas_call(
        _attn_body,
        grid=(_N // _R3,),
        in_specs=[
            pl.BlockSpec((_R3, _K * _C), lambda b: (b, 0)),
            pl.BlockSpec((_R3, _C), lambda b: (b, 0)),
            pl.BlockSpec((1, _C), lambda b: (0, 0)),
            pl.BlockSpec((1, _C), lambda b: (0, 0)),
        ],
        out_specs=pl.BlockSpec((_R3, _C), lambda b: (b, 0)),
        out_shape=jax.ShapeDtypeStruct((_N, _C), jnp.float32),
    )(g2, hr, att2, bias2)


def kernel(x, Wl, Wr, att, bias):
    xt = x.T
    hl, hr, idx = _stage1(x, xt, Wl, Wr)
    g = _sc_gather(hl, idx.reshape(-1))
    g2 = g.reshape(_N, _K * _C)
    out = _stage3(g2, hr, att.reshape(1, _C), bias.reshape(1, _C))
    return out


# speedup vs baseline: 7.1187x; 1.0768x over previous
import functools

import jax
import jax.numpy as jnp
from jax import lax
from jax.experimental import pallas as pl
from jax.experimental.pallas import tpu as pltpu
from jax.experimental.pallas import tpu_sc as plsc

_N = 10000
_D = 256
_C = 256
_K = 16

_R1 = 80
_R3 = 200
_GBLK = 128


def _sqh_body(xt_ref, out_ref):
    xt = xt_ref[...]
    out_ref[...] = 0.5 * jnp.sum(xt * xt, axis=0, keepdims=True)


def _sqh(xt):
    return pl.pallas_call(
        _sqh_body,
        out_shape=jax.ShapeDtypeStruct((1, _N), jnp.float32),
    )(xt)


def _knn_proj_body(x_blk, xt_ref, wl_ref, wr_ref, sqh_ref, hl_out, hr_out,
                   idx_out):
    rows = x_blk[...]
    b = pl.program_id(0)
    hl_out[...] = jnp.dot(rows, wl_ref[...], preferred_element_type=jnp.float32)
    hr_out[...] = jnp.dot(rows, wr_ref[...], preferred_element_type=jnp.float32)

    xt = xt_ref[...]
    prod = jnp.dot(rows, xt, preferred_element_type=jnp.float32)
    d2 = sqh_ref[...] - prod

    cols = lax.broadcasted_iota(jnp.int32, (_R1, _N), 1).astype(jnp.float32)
    row_ids = (b * _R1 + lax.broadcasted_iota(jnp.int32, (_R1, 1), 0)
               ).astype(jnp.float32)
    d2 = jnp.where(cols == row_ids, jnp.inf, d2)

    picks = []
    for _ in range(_K):
        m = jnp.min(d2, axis=1, keepdims=True)
        hit = d2 == m
        picks.append(jnp.min(jnp.where(hit, cols, jnp.float32(_N)),
                             axis=1, keepdims=True))
        d2 = jnp.where(hit, jnp.inf, d2)
    idx_out[...] = jnp.concatenate(picks, axis=1).astype(jnp.int32)


def _stage1(x, xt, wl, wr, sqh):
    return pl.pallas_call(
        _knn_proj_body,
        grid=(_N // _R1,),
        in_specs=[
            pl.BlockSpec((_R1, _D), lambda b: (b, 0)),
            pl.BlockSpec((_D, _N), lambda b: (0, 0)),
            pl.BlockSpec((_D, _C), lambda b: (0, 0)),
            pl.BlockSpec((_D, _C), lambda b: (0, 0)),
            pl.BlockSpec((1, _N), lambda b: (0, 0)),
        ],
        out_specs=[
            pl.BlockSpec((_R1, _C), lambda b: (b, 0)),
            pl.BlockSpec((_R1, _C), lambda b: (b, 0)),
            pl.BlockSpec((_R1, _K), lambda b: (b, 0)),
        ],
        out_shape=[
            jax.ShapeDtypeStruct((_N, _C), jnp.float32),
            jax.ShapeDtypeStruct((_N, _C), jnp.float32),
            jax.ShapeDtypeStruct((_N, _K), jnp.int32),
        ],
    )(x, xt, wl, wr, sqh)


_NBLOCKS = (_N * _K) // _GBLK
_NW = 32


def _sc_gather(hl, idx_flat):
    mesh = plsc.VectorSubcoreMesh(core_axis_name="c", subcore_axis_name="s")

    @functools.partial(
        pl.kernel,
        mesh=mesh,
        out_type=jax.ShapeDtypeStruct((_N * _K, _C), jnp.float32),
        scratch_types=[
            pltpu.VMEM((_GBLK,), jnp.int32),
            pltpu.VMEM((_GBLK, _C), jnp.float32),
            pltpu.SemaphoreType.DMA,
        ],
    )
    def gather_kernel(hl_hbm, idx_hbm, out_hbm, idx_v, rows_v, sem):
        cid = lax.axis_index("c")
        sid = lax.axis_index("s")
        w = sid * 2 + cid
        nb_w = (_NBLOCKS - w + _NW - 1) // _NW

        def body(i, carry):
            blk = w + i * _NW
            pltpu.sync_copy(idx_hbm.at[pl.ds(blk * _GBLK, _GBLK)], idx_v)
            pltpu.async_copy(hl_hbm.at[idx_v], rows_v, sem).wait()
            pltpu.sync_copy(rows_v, out_hbm.at[pl.ds(blk * _GBLK, _GBLK)])
            return carry

        lax.fori_loop(0, nb_w, body, 0)

    return gather_kernel(hl, idx_flat)


def _attn_body(g_ref, hr_ref, att_ref, bias_ref, out_ref):
    g = g_ref[...]
    hr = hr_ref[...]
    a = att_ref[...]
    es = []
    for k in range(_K):
        gk = g[:, k * _C:(k + 1) * _C]
        z = gk + hr
        z = jnp.where(z > 0, z, 0.2 * z)
        es.append(jnp.sum(z * a, axis=1, keepdims=True))
    e = jnp.concatenate(es, axis=1)
    m = jnp.max(e, axis=1, keepdims=True)
    ez = jnp.exp(e - m)
    denom = jnp.sum(ez, axis=1, keepdims=True)
    alpha = ez / denom
    acc = alpha[:, 0:1] * g[:, 0:_C]
    for k in range(1, _K):
        acc = acc + alpha[:, k:k + 1] * g[:, k * _C:(k + 1) * _C]
    out_ref[...] = acc + bias_ref[...]


def _stage3(g2, hr, att2, bias2):
    return pl.pallas_call(
        _attn_body,
        grid=(_N // _R3,),
        in_specs=[
            pl.BlockSpec((_R3, _K * _C), lambda b: (b, 0)),
            pl.BlockSpec((_R3, _C), lambda b: (b, 0)),
            pl.BlockSpec((1, _C), lambda b: (0, 0)),
            pl.BlockSpec((1, _C), lambda b: (0, 0)),
        ],
        out_specs=pl.BlockSpec((_R3, _C), lambda b: (b, 0)),
        out_shape=jax.ShapeDtypeStruct((_N, _C), jnp.float32),
    )(g2, hr, att2, bias2)


def kernel(x, Wl, Wr, att, bias):
    xt = x.T
    hl, hr, idx = _stage1(x, xt, Wl, Wr, _sqh(xt))
    g = _sc_gather(hl, idx.reshape(-1))
    g2 = g.reshape(_N, _K * _C)
    out = _stage3(g2, hr, att.reshape(1, _C), bias.reshape(1, _C))
    return out


# speedup vs baseline: 8.0116x; 1.1254x over previous
import functools

import jax
import jax.numpy as jnp
from jax import lax
from jax.experimental import pallas as pl
from jax.experimental.pallas import tpu as pltpu
from jax.experimental.pallas import tpu_sc as plsc

_N = 10000
_D = 256
_C = 256
_K = 16

_G = 1250
_NC = 128
_R1 = 80
_R2 = 200
_R3 = 200
_NW = 32


def _sqh_body(xt_ref, out_ref):
    xt = xt_ref[...]
    out_ref[...] = 0.5 * jnp.sum(xt * xt, axis=0, keepdims=True)


def _sqh(xt):
    return pl.pallas_call(
        _sqh_body,
        out_shape=jax.ShapeDtypeStruct((1, _N), jnp.float32),
    )(xt)


def _knn_proj_body(x_blk, xt_ref, wl_ref, wr_ref, sqh_ref, hl_out, hr_out,
                   ci_out, d2_out):
    rows = x_blk[...]
    b = pl.program_id(0)
    hl_out[...] = jnp.dot(rows, wl_ref[...], preferred_element_type=jnp.float32)
    hr_out[...] = jnp.dot(rows, wr_ref[...], preferred_element_type=jnp.float32)

    prod = jnp.dot(rows, xt_ref[...], preferred_element_type=jnp.float32)
    d2 = sqh_ref[...] - prod

    cols = lax.broadcasted_iota(jnp.int32, (_R1, _N), 1)
    row_ids = b * _R1 + lax.broadcasted_iota(jnp.int32, (_R1, 1), 0)
    d2 = jnp.where(cols == row_ids, jnp.inf, d2)
    d2_out[...] = d2

    r = d2[:, 0:_G]
    for j in range(1, 8):
        r = jnp.minimum(r, d2[:, j * _G:(j + 1) * _G])

    gcols = lax.broadcasted_iota(jnp.int32, (_R1, _G), 1).astype(jnp.float32)
    picks = []
    for _ in range(_K):
        m = jnp.min(r, axis=1, keepdims=True)
        hit = r == m
        picks.append(jnp.min(jnp.where(hit, gcols, jnp.float32(_G)),
                             axis=1, keepdims=True))
        r = jnp.where(hit, jnp.inf, r)
    qsel = jnp.concatenate(picks, axis=1).astype(jnp.int32)

    ci = jnp.concatenate([qsel + j * _G for j in range(8)], axis=1)
    ci_out[...] = ci + row_ids * _N


def _stage1(x, xt, wl, wr, sqh):
    return pl.pallas_call(
        _knn_proj_body,
        grid=(_N // _R1,),
        in_specs=[
            pl.BlockSpec((_R1, _D), lambda b: (b, 0)),
            pl.BlockSpec((_D, _N), lambda b: (0, 0)),
            pl.BlockSpec((_D, _C), lambda b: (0, 0)),
            pl.BlockSpec((_D, _C), lambda b: (0, 0)),
            pl.BlockSpec((1, _N), lambda b: (0, 0)),
        ],
        out_specs=[
            pl.BlockSpec((_R1, _C), lambda b: (b, 0)),
            pl.BlockSpec((_R1, _C), lambda b: (b, 0)),
            pl.BlockSpec((_R1, _NC), lambda b: (b, 0)),
            pl.BlockSpec((_R1, _N), lambda b: (b, 0)),
        ],
        out_shape=[
            jax.ShapeDtypeStruct((_N, _C), jnp.float32),
            jax.ShapeDtypeStruct((_N, _C), jnp.float32),
            jax.ShapeDtypeStruct((_N, _NC), jnp.int32),
            jax.ShapeDtypeStruct((_N, _N), jnp.float32),
        ],
    )(x, xt, wl, wr, sqh)


_NCH = (_N * _NC) // 128
_TA = (_NCH + _NW - 1) // _NW


def _sc_cand_gather(d2flat, ci_flat):
    mesh = plsc.VectorSubcoreMesh(core_axis_name="c", subcore_axis_name="s")

    @functools.partial(
        pl.kernel,
        mesh=mesh,
        out_type=jax.ShapeDtypeStruct((_N * _NC,), jnp.float32),
        scratch_types=[
            pltpu.VMEM((128,), jnp.int32),
            pltpu.VMEM((128,), jnp.float32),
            pltpu.SemaphoreType.DMA,
        ],
    )
    def body(d2_hbm, ci_hbm, out_hbm, idx_v, val_v, sem):
        cid = lax.axis_index("c")
        sid = lax.axis_index("s")
        w = sid * 2 + cid

        def do(t, carry):
            ch = jnp.minimum(w + t * _NW, _NCH - 1)
            off = ch * 128
            pltpu.sync_copy(ci_hbm.at[pl.ds(off, 128)], idx_v)
            pltpu.async_copy(d2_hbm.at[idx_v], val_v, sem).wait()
            pltpu.sync_copy(val_v, out_hbm.at[pl.ds(off, 128)])
            return carry

        lax.fori_loop(0, _TA, do, 0)

    return body(d2flat, ci_flat)


def _select_body(vals_ref, ci_ref, idx_out):
    b = pl.program_id(0)
    v = vals_ref[...]
    row_ids = b * _R2 + lax.broadcasted_iota(jnp.int32, (_R2, 1), 0)
    colsf = (ci_ref[...] - row_ids * _N).astype(jnp.float32)
    picks = []
    for _ in range(_K):
        m = jnp.min(v, axis=1, keepdims=True)
        hit = v == m
        picks.append(jnp.min(jnp.where(hit, colsf, jnp.float32(_N)),
                             axis=1, keepdims=True))
        v = jnp.where(hit, jnp.inf, v)
    idx_out[...] = jnp.concatenate(picks, axis=1).astype(jnp.int32)


def _stage_select(vals, ci):
    return pl.pallas_call(
        _select_body,
        grid=(_N // _R2,),
        in_specs=[
            pl.BlockSpec((_R2, _NC), lambda b: (b, 0)),
            pl.BlockSpec((_R2, _NC), lambda b: (b, 0)),
        ],
        out_specs=pl.BlockSpec((_R2, _K), lambda b: (b, 0)),
        out_shape=jax.ShapeDtypeStruct((_N, _K), jnp.int32),
    )(vals, ci)


_GBLK = 128
_NBLOCKS = (_N * _K) // _GBLK


def _sc_row_gather(hl, idx_flat):
    mesh = plsc.VectorSubcoreMesh(core_axis_name="c", subcore_axis_name="s")

    @functools.partial(
        pl.kernel,
        mesh=mesh,
        out_type=jax.ShapeDtypeStruct((_N * _K, _C), jnp.float32),
        scratch_types=[
            pltpu.VMEM((_GBLK,), jnp.int32),
            pltpu.VMEM((_GBLK, _C), jnp.float32),
            pltpu.SemaphoreType.DMA,
        ],
    )
    def body(hl_hbm, idx_hbm, out_hbm, idx_v, rows_v, sem):
        cid = lax.axis_index("c")
        sid = lax.axis_index("s")
        w = sid * 2 + cid
        nb_w = (_NBLOCKS - w + _NW - 1) // _NW

        def do(i, carry):
            blk = w + i * _NW
            pltpu.sync_copy(idx_hbm.at[pl.ds(blk * _GBLK, _GBLK)], idx_v)
            pltpu.async_copy(hl_hbm.at[idx_v], rows_v, sem).wait()
            pltpu.sync_copy(rows_v, out_hbm.at[pl.ds(blk * _GBLK, _GBLK)])
            return carry

        lax.fori_loop(0, nb_w, do, 0)

    return body(hl, idx_flat)


def _attn_body(g_ref, hr_ref, att_ref, bias_ref, out_ref):
    g = g_ref[...]
    hr = hr_ref[...]
    a = att_ref[...]
    es = []
    for k in range(_K):
        gk = g[:, k * _C:(k + 1) * _C]
        z = gk + hr
        z = jnp.where(z > 0, z, 0.2 * z)
        es.append(jnp.sum(z * a, axis=1, keepdims=True))
    e = jnp.concatenate(es, axis=1)
    m = jnp.max(e, axis=1, keepdims=True)
    ez = jnp.exp(e - m)
    denom = jnp.sum(ez, axis=1, keepdims=True)
    alpha = ez / denom
    acc = alpha[:, 0:1] * g[:, 0:_C]
    for k in range(1, _K):
        acc = acc + alpha[:, k:k + 1] * g[:, k * _C:(k + 1) * _C]
    out_ref[...] = acc + bias_ref[...]


def _stage_attn(g2, hr, att2, bias2):
    return pl.pallas_call(
        _attn_body,
        grid=(_N // _R3,),
        in_specs=[
            pl.BlockSpec((_R3, _K * _C), lambda b: (b, 0)),
            pl.BlockSpec((_R3, _C), lambda b: (b, 0)),
            pl.BlockSpec((1, _C), lambda b: (0, 0)),
            pl.BlockSpec((1, _C), lambda b: (0, 0)),
        ],
        out_specs=pl.BlockSpec((_R3, _C), lambda b: (b, 0)),
        out_shape=jax.ShapeDtypeStruct((_N, _C), jnp.float32),
    )(g2, hr, att2, bias2)


def kernel(x, Wl, Wr, att, bias):
    xt = x.T
    hl, hr, ci, d2 = _stage1(x, xt, Wl, Wr, _sqh(xt))
    vals = _sc_cand_gather(d2.reshape(-1), ci.reshape(-1))
    idx = _stage_select(vals.reshape(_N, _NC), ci)
    g = _sc_row_gather(hl, idx.reshape(-1))
    g2 = g.reshape(_N, _K * _C)
    return _stage_attn(g2, hr, att.reshape(1, _C), bias.reshape(1, _C))


# speedup vs baseline: 9.7025x; 1.2111x over previous
import functools

import jax
import jax.numpy as jnp
from jax import lax
from jax.experimental import pallas as pl
from jax.experimental.pallas import tpu as pltpu
from jax.experimental.pallas import tpu_sc as plsc

_N = 10000
_D = 256
_C = 256
_K = 16

_G = 1250
_NC = 128
_R1 = 80
_R2 = 200
_R3 = 200
_NW = 32


def _sqh_body(xt_ref, out_ref):
    xt = xt_ref[...]
    out_ref[...] = 0.5 * jnp.sum(xt * xt, axis=0, keepdims=True)


def _sqh(xt):
    return pl.pallas_call(
        _sqh_body,
        out_shape=jax.ShapeDtypeStruct((1, _N), jnp.float32),
    )(xt)


def _knn_proj_body(x_blk, xt_ref, wl_ref, wr_ref, sqh_ref, hl_out, hr_out,
                   ci_out, d2_out):
    rows = x_blk[...]
    b = pl.program_id(0)
    hl_out[...] = jnp.dot(rows, wl_ref[...], preferred_element_type=jnp.float32)
    hr_out[...] = jnp.dot(rows, wr_ref[...], preferred_element_type=jnp.float32)

    prod = jnp.dot(rows, xt_ref[...], preferred_element_type=jnp.float32)
    d2 = sqh_ref[...] - prod

    cols = lax.broadcasted_iota(jnp.int32, (_R1, _N), 1)
    row_ids = b * _R1 + lax.broadcasted_iota(jnp.int32, (_R1, 1), 0)
    d2 = jnp.where(cols == row_ids, jnp.inf, d2)
    d2_out[...] = d2

    r = d2[:, 0:_G]
    for j in range(1, 8):
        r = jnp.minimum(r, d2[:, j * _G:(j + 1) * _G])

    gcols = lax.broadcasted_iota(jnp.int32, (_R1, _G), 1).astype(jnp.float32)
    picks = []
    for _ in range(_K):
        m = jnp.min(r, axis=1, keepdims=True)
        hit = r == m
        picks.append(jnp.min(jnp.where(hit, gcols, jnp.float32(_G)),
                             axis=1, keepdims=True))
        r = jnp.where(hit, jnp.inf, r)
    qsel = jnp.concatenate(picks, axis=1).astype(jnp.int32)

    ci = jnp.concatenate([qsel + j * _G for j in range(8)], axis=1)
    ci_out[...] = ci + row_ids * _N


def _stage1(x, xt, wl, wr, sqh):
    return pl.pallas_call(
        _knn_proj_body,
        grid=(_N // _R1,),
        in_specs=[
            pl.BlockSpec((_R1, _D), lambda b: (b, 0)),
            pl.BlockSpec((_D, _N), lambda b: (0, 0)),
            pl.BlockSpec((_D, _C), lambda b: (0, 0)),
            pl.BlockSpec((_D, _C), lambda b: (0, 0)),
            pl.BlockSpec((1, _N), lambda b: (0, 0)),
        ],
        out_specs=[
            pl.BlockSpec((_R1, _C), lambda b: (b, 0)),
            pl.BlockSpec((_R1, _C), lambda b: (b, 0)),
            pl.BlockSpec((_R1, _NC), lambda b: (b, 0)),
            pl.BlockSpec((_R1, _N), lambda b: (b, 0)),
        ],
        out_shape=[
            jax.ShapeDtypeStruct((_N, _C), jnp.float32),
            jax.ShapeDtypeStruct((_N, _C), jnp.float32),
            jax.ShapeDtypeStruct((_N, _NC), jnp.int32),
            jax.ShapeDtypeStruct((_N, _N), jnp.float32),
        ],
    )(x, xt, wl, wr, sqh)


_BATCH_A = 1024
_NBAT = (_N * _NC) // _BATCH_A
_TA = (_NBAT + _NW - 1) // _NW


def _sc_cand_gather(d2flat, ci_flat):
    mesh = plsc.VectorSubcoreMesh(core_axis_name="c", subcore_axis_name="s")

    @functools.partial(
        pl.kernel,
        mesh=mesh,
        out_type=jax.ShapeDtypeStruct((_N * _NC,), jnp.float32),
        scratch_types=[
            pltpu.VMEM((_BATCH_A,), jnp.int32),
            pltpu.VMEM((_BATCH_A,), jnp.float32),
            pltpu.SemaphoreType.DMA,
        ],
    )
    def body(d2_hbm, ci_hbm, out_hbm, idx_v, val_v, sem):
        cid = lax.axis_index("c")
        sid = lax.axis_index("s")
        w = sid * 2 + cid

        def do(t, carry):
            b = jnp.minimum(w + t * _NW, _NBAT - 1)
            off = b * _BATCH_A
            pltpu.sync_copy(ci_hbm.at[pl.ds(off, _BATCH_A)], idx_v)
            handles = []
            for k in range(_BATCH_A // 128):
                handles.append(pltpu.async_copy(
                    d2_hbm.at[idx_v.at[pl.ds(k * 128, 128)]],
                    val_v.at[pl.ds(k * 128, 128)], sem))
            for h in handles:
                h.wait()
            pltpu.sync_copy(val_v, out_hbm.at[pl.ds(off, _BATCH_A)])
            return carry

        lax.fori_loop(0, _TA, do, 0)

    return body(d2flat, ci_flat)


def _select_body(vals_ref, ci_ref, idx_out):
    b = pl.program_id(0)
    v = vals_ref[...]
    row_ids = b * _R2 + lax.broadcasted_iota(jnp.int32, (_R2, 1), 0)
    colsf = (ci_ref[...] - row_ids * _N).astype(jnp.float32)
    picks = []
    for _ in range(_K):
        m = jnp.min(v, axis=1, keepdims=True)
        hit = v == m
        picks.append(jnp.min(jnp.where(hit, colsf, jnp.float32(_N)),
                             axis=1, keepdims=True))
        v = jnp.where(hit, jnp.inf, v)
    idx_out[...] = jnp.concatenate(picks, axis=1).astype(jnp.int32)


def _stage_select(vals, ci):
    return pl.pallas_call(
        _select_body,
        grid=(_N // _R2,),
        in_specs=[
            pl.BlockSpec((_R2, _NC), lambda b: (b, 0)),
            pl.BlockSpec((_R2, _NC), lambda b: (b, 0)),
        ],
        out_specs=pl.BlockSpec((_R2, _K), lambda b: (b, 0)),
        out_shape=jax.ShapeDtypeStruct((_N, _K), jnp.int32),
    )(vals, ci)


_GBLK = 128
_NBLOCKS = (_N * _K) // _GBLK


def _sc_row_gather(hl, idx_flat):
    mesh = plsc.VectorSubcoreMesh(core_axis_name="c", subcore_axis_name="s")

    @functools.partial(
        pl.kernel,
        mesh=mesh,
        out_type=jax.ShapeDtypeStruct((_N * _K, _C), jnp.float32),
        scratch_types=[
            pltpu.VMEM((_GBLK,), jnp.int32),
            pltpu.VMEM((_GBLK,), jnp.int32),
            pltpu.VMEM((_GBLK, _C), jnp.float32),
            pltpu.VMEM((_GBLK, _C), jnp.float32),
            pltpu.SemaphoreType.DMA,
            pltpu.SemaphoreType.DMA,
        ],
    )
    def body(hl_hbm, idx_hbm, out_hbm, idx0, idx1, rows0, rows1, semg, semw):
        cid = lax.axis_index("c")
        sid = lax.axis_index("s")
        w = sid * 2 + cid

        def do(i, carry):
            ba = jnp.minimum(w + (2 * i) * _NW, _NBLOCKS - 1)
            bb = jnp.minimum(w + (2 * i + 1) * _NW, _NBLOCKS - 1)
            pltpu.sync_copy(idx_hbm.at[pl.ds(ba * _GBLK, _GBLK)], idx0)
            ga = pltpu.async_copy(hl_hbm.at[idx0], rows0, semg)
            pltpu.sync_copy(idx_hbm.at[pl.ds(bb * _GBLK, _GBLK)], idx1)
            ga.wait()
            gb = pltpu.async_copy(hl_hbm.at[idx1], rows1, semg)
            wa = pltpu.async_copy(rows0, out_hbm.at[pl.ds(ba * _GBLK, _GBLK)],
                                  semw)
            gb.wait()
            wb = pltpu.async_copy(rows1, out_hbm.at[pl.ds(bb * _GBLK, _GBLK)],
                                  semw)
            wa.wait()
            wb.wait()
            return carry

        lax.fori_loop(0, (_NBLOCKS // _NW + 2) // 2, do, 0)

    return body(hl, idx_flat)


def _attn_body(g_ref, hr_ref, att_ref, bias_ref, out_ref):
    g = g_ref[...]
    hr = hr_ref[...]
    a = att_ref[...]
    es = []
    for k in range(_K):
        gk = g[:, k * _C:(k + 1) * _C]
        z = gk + hr
        z = jnp.where(z > 0, z, 0.2 * z)
        es.append(jnp.sum(z * a, axis=1, keepdims=True))
    e = jnp.concatenate(es, axis=1)
    m = jnp.max(e, axis=1, keepdims=True)
    ez = jnp.exp(e - m)
    denom = jnp.sum(ez, axis=1, keepdims=True)
    alpha = ez / denom
    acc = alpha[:, 0:1] * g[:, 0:_C]
    for k in range(1, _K):
        acc = acc + alpha[:, k:k + 1] * g[:, k * _C:(k + 1) * _C]
    out_ref[...] = acc + bias_ref[...]


def _stage_attn(g2, hr, att2, bias2):
    return pl.pallas_call(
        _attn_body,
        grid=(_N // _R3,),
        in_specs=[
            pl.BlockSpec((_R3, _K * _C), lambda b: (b, 0)),
            pl.BlockSpec((_R3, _C), lambda b: (b, 0)),
            pl.BlockSpec((1, _C), lambda b: (0, 0)),
            pl.BlockSpec((1, _C), lambda b: (0, 0)),
        ],
        out_specs=pl.BlockSpec((_R3, _C), lambda b: (b, 0)),
        out_shape=jax.ShapeDtypeStruct((_N, _C), jnp.float32),
    )(g2, hr, att2, bias2)


def kernel(x, Wl, Wr, att, bias):
    xt = x.T
    hl, hr, ci, d2 = _stage1(x, xt, Wl, Wr, _sqh(xt))
    vals = _sc_cand_gather(d2.reshape(-1), ci.reshape(-1))
    idx = _stage_select(vals.reshape(_N, _NC), ci)
    g = _sc_row_gather(hl, idx.reshape(-1))
    g2 = g.reshape(_N, _K * _C)
    return _stage_attn(g2, hr, att.reshape(1, _C), bias.reshape(1, _C))


# speedup vs baseline: 9.9069x; 1.0211x over previous
import functools

import jax
import jax.numpy as jnp
from jax import lax
from jax.experimental import pallas as pl
from jax.experimental.pallas import tpu as pltpu
from jax.experimental.pallas import tpu_sc as plsc

_N = 10000
_D = 256
_C = 256
_K = 16

_G = 1250
_NC = 128
_R1 = 200
_R2 = 200
_R3 = 200
_NW = 32


def _sqh_body(xt_ref, out_ref):
    xt = xt_ref[...]
    out_ref[...] = 0.5 * jnp.sum(xt * xt, axis=0, keepdims=True)


def _sqh(xt):
    return pl.pallas_call(
        _sqh_body,
        out_shape=jax.ShapeDtypeStruct((1, _N), jnp.float32),
    )(xt)


def _knn_proj_body(x_blk, xt_ref, wl_ref, wr_ref, sqh_ref, hl_out, hr_out,
                   ci_out, d2_out):
    rows = x_blk[...]
    b = pl.program_id(0)
    hl_out[...] = jnp.dot(rows, wl_ref[...], preferred_element_type=jnp.float32)
    hr_out[...] = jnp.dot(rows, wr_ref[...], preferred_element_type=jnp.float32)

    prod = jnp.dot(rows, xt_ref[...], preferred_element_type=jnp.float32)
    d2 = sqh_ref[...] - prod

    cols = lax.broadcasted_iota(jnp.int32, (_R1, _N), 1)
    row_ids = b * _R1 + lax.broadcasted_iota(jnp.int32, (_R1, 1), 0)
    d2 = jnp.where(cols == row_ids, jnp.inf, d2)
    d2_out[...] = d2

    r = d2[:, 0:_G]
    for j in range(1, 8):
        r = jnp.minimum(r, d2[:, j * _G:(j + 1) * _G])

    gcols = lax.broadcasted_iota(jnp.int32, (_R1, _G), 1).astype(jnp.float32)
    picks = []
    for _ in range(_K):
        m = jnp.min(r, axis=1, keepdims=True)
        hit = r == m
        picks.append(jnp.min(jnp.where(hit, gcols, jnp.float32(_G)),
                             axis=1, keepdims=True))
        r = jnp.where(hit, jnp.inf, r)
    qsel = jnp.concatenate(picks, axis=1).astype(jnp.int32)

    ci = jnp.concatenate([qsel + j * _G for j in range(8)], axis=1)
    ci_out[...] = ci + row_ids * _N


def _stage1(x, xt, wl, wr, sqh):
    return pl.pallas_call(
        _knn_proj_body,
        grid=(_N // _R1,),
        in_specs=[
            pl.BlockSpec((_R1, _D), lambda b: (b, 0)),
            pl.BlockSpec((_D, _N), lambda b: (0, 0)),
            pl.BlockSpec((_D, _C), lambda b: (0, 0)),
            pl.BlockSpec((_D, _C), lambda b: (0, 0)),
            pl.BlockSpec((1, _N), lambda b: (0, 0)),
        ],
        out_specs=[
            pl.BlockSpec((_R1, _C), lambda b: (b, 0)),
            pl.BlockSpec((_R1, _C), lambda b: (b, 0)),
            pl.BlockSpec((_R1, _NC), lambda b: (b, 0)),
            pl.BlockSpec((_R1, _N), lambda b: (b, 0)),
        ],
        out_shape=[
            jax.ShapeDtypeStruct((_N, _C), jnp.float32),
            jax.ShapeDtypeStruct((_N, _C), jnp.float32),
            jax.ShapeDtypeStruct((_N, _NC), jnp.int32),
            jax.ShapeDtypeStruct((_N, _N), jnp.float32),
        ],
    )(x, xt, wl, wr, sqh)


_BATCH_A = 1024
_NBAT = (_N * _NC) // _BATCH_A
_TA = (_NBAT + _NW - 1) // _NW


def _sc_cand_gather(d2flat, ci_flat):
    mesh = plsc.VectorSubcoreMesh(core_axis_name="c", subcore_axis_name="s")

    @functools.partial(
        pl.kernel,
        mesh=mesh,
        out_type=jax.ShapeDtypeStruct((_N * _NC,), jnp.float32),
        scratch_types=[
            pltpu.VMEM((_BATCH_A,), jnp.int32),
            pltpu.VMEM((_BATCH_A,), jnp.float32),
            pltpu.SemaphoreType.DMA,
        ],
    )
    def body(d2_hbm, ci_hbm, out_hbm, idx_v, val_v, sem):
        cid = lax.axis_index("c")
        sid = lax.axis_index("s")
        w = sid * 2 + cid

        def do(t, carry):
            b = jnp.minimum(w + t * _NW, _NBAT - 1)
            off = b * _BATCH_A
            pltpu.sync_copy(ci_hbm.at[pl.ds(off, _BATCH_A)], idx_v)
            handles = []
            for k in range(_BATCH_A // 128):
                handles.append(pltpu.async_copy(
                    d2_hbm.at[idx_v.at[pl.ds(k * 128, 128)]],
                    val_v.at[pl.ds(k * 128, 128)], sem))
            for h in handles:
                h.wait()
            pltpu.sync_copy(val_v, out_hbm.at[pl.ds(off, _BATCH_A)])
            return carry

        lax.fori_loop(0, _TA, do, 0)

    return body(d2flat, ci_flat)


def _select_body(vals_ref, ci_ref, idx_out):
    b = pl.program_id(0)
    v = vals_ref[...]
    row_ids = b * _R2 + lax.broadcasted_iota(jnp.int32, (_R2, 1), 0)
    colsf = (ci_ref[...] - row_ids * _N).astype(jnp.float32)
    picks = []
    for _ in range(_K):
        m = jnp.min(v, axis=1, keepdims=True)
        hit = v == m
        picks.append(jnp.min(jnp.where(hit, colsf, jnp.float32(_N)),
                             axis=1, keepdims=True))
        v = jnp.where(hit, jnp.inf, v)
    idx_out[...] = jnp.concatenate(picks, axis=1).astype(jnp.int32)


def _stage_select(vals, ci):
    return pl.pallas_call(
        _select_body,
        grid=(_N // _R2,),
        in_specs=[
            pl.BlockSpec((_R2, _NC), lambda b: (b, 0)),
            pl.BlockSpec((_R2, _NC), lambda b: (b, 0)),
        ],
        out_specs=pl.BlockSpec((_R2, _K), lambda b: (b, 0)),
        out_shape=jax.ShapeDtypeStruct((_N, _K), jnp.int32),
    )(vals, ci)


_GBLK = 128
_NBLOCKS = (_N * _K) // _GBLK


def _sc_row_gather(hl, idx_flat):
    mesh = plsc.VectorSubcoreMesh(core_axis_name="c", subcore_axis_name="s")

    @functools.partial(
        pl.kernel,
        mesh=mesh,
        out_type=jax.ShapeDtypeStruct((_N * _K, _C), jnp.float32),
        scratch_types=[
            pltpu.VMEM((_GBLK,), jnp.int32),
            pltpu.VMEM((_GBLK,), jnp.int32),
            pltpu.VMEM((_GBLK, _C), jnp.float32),
            pltpu.VMEM((_GBLK, _C), jnp.float32),
            pltpu.SemaphoreType.DMA,
            pltpu.SemaphoreType.DMA,
        ],
    )
    def body(hl_hbm, idx_hbm, out_hbm, idx0, idx1, rows0, rows1, semg, semw):
        cid = lax.axis_index("c")
        sid = lax.axis_index("s")
        w = sid * 2 + cid

        def do(i, carry):
            ba = jnp.minimum(w + (2 * i) * _NW, _NBLOCKS - 1)
            bb = jnp.minimum(w + (2 * i + 1) * _NW, _NBLOCKS - 1)
            pltpu.sync_copy(idx_hbm.at[pl.ds(ba * _GBLK, _GBLK)], idx0)
            ga = pltpu.async_copy(hl_hbm.at[idx0], rows0, semg)
            pltpu.sync_copy(idx_hbm.at[pl.ds(bb * _GBLK, _GBLK)], idx1)
            ga.wait()
            gb = pltpu.async_copy(hl_hbm.at[idx1], rows1, semg)
            wa = pltpu.async_copy(rows0, out_hbm.at[pl.ds(ba * _GBLK, _GBLK)],
                                  semw)
            gb.wait()
            wb = pltpu.async_copy(rows1, out_hbm.at[pl.ds(bb * _GBLK, _GBLK)],
                                  semw)
            wa.wait()
            wb.wait()
            return carry

        lax.fori_loop(0, (_NBLOCKS // _NW + 2) // 2, do, 0)

    return body(hl, idx_flat)


def _attn_body(g_ref, hr_ref, att_ref, bias_ref, out_ref):
    g = g_ref[...]
    hr = hr_ref[...]
    a = att_ref[...]
    es = []
    for k in range(_K):
        gk = g[:, k * _C:(k + 1) * _C]
        z = gk + hr
        z = jnp.where(z > 0, z, 0.2 * z)
        es.append(jnp.sum(z * a, axis=1, keepdims=True))
    e = jnp.concatenate(es, axis=1)
    m = jnp.max(e, axis=1, keepdims=True)
    ez = jnp.exp(e - m)
    denom = jnp.sum(ez, axis=1, keepdims=True)
    alpha = ez / denom
    acc = alpha[:, 0:1] * g[:, 0:_C]
    for k in range(1, _K):
        acc = acc + alpha[:, k:k + 1] * g[:, k * _C:(k + 1) * _C]
    out_ref[...] = acc + bias_ref[...]


def _stage_attn(g2, hr, att2, bias2):
    return pl.pallas_call(
        _attn_body,
        grid=(_N // _R3,),
        in_specs=[
            pl.BlockSpec((_R3, _K * _C), lambda b: (b, 0)),
            pl.BlockSpec((_R3, _C), lambda b: (b, 0)),
            pl.BlockSpec((1, _C), lambda b: (0, 0)),
            pl.BlockSpec((1, _C), lambda b: (0, 0)),
        ],
        out_specs=pl.BlockSpec((_R3, _C), lambda b: (b, 0)),
        out_shape=jax.ShapeDtypeStruct((_N, _C), jnp.float32),
    )(g2, hr, att2, bias2)


def kernel(x, Wl, Wr, att, bias):
    xt = x.T
    hl, hr, ci, d2 = _stage1(x, xt, Wl, Wr, _sqh(xt))
    vals = _sc_cand_gather(d2.reshape(-1), ci.reshape(-1))
    idx = _stage_select(vals.reshape(_N, _NC), ci)
    g = _sc_row_gather(hl, idx.reshape(-1))
    g2 = g.reshape(_N, _K * _C)
    return _stage_attn(g2, hr, att.reshape(1, _C), bias.reshape(1, _C))
